# Initial kernel scaffold; baseline (speedup 1.0000x reference)
import jax
import jax.numpy as jnp
from jax.experimental import pallas as pl


def kernel(x, edge_index, W1, b1, W2, b2):
    raise NotImplementedError("write your pallas kernel here")


# speedup vs baseline: 213.9863x; 213.9863x over previous
import functools

import jax
import jax.numpy as jnp
from jax import lax
from jax.experimental import pallas as pl
from jax.experimental.pallas import tpu as pltpu
from jax.experimental.pallas import tpu_sc as plsc

_NC = 2
_NS = 16
_NW = _NC * _NS
_B = 8000


def _round_up(a, b):
  return -(-a // b) * b


def _deg_body(np_, epw, nchunks, dst_hbm, out_hbm,
              acc_sh, dst_v, ones_v):
  c = lax.axis_index("c")
  s = lax.axis_index("s")
  rpt = np_ // _NS
  sl = pl.ds(s * rpt, rpt)
  stage = ones_v.at[pl.ds(0, rpt)]

  @pl.loop(0, rpt // 16)
  def _(i):
    ones_v[pl.ds(i * 16, 16)] = jnp.zeros((16,), jnp.float32)

  pltpu.sync_copy(stage, acc_sh.at[sl])

  @pl.loop(0, _B // 16)
  def _(i):
    ones_v[pl.ds(i * 16, 16)] = jnp.full((16,), 1.0, jnp.float32)

  plsc.subcore_barrier()
  wid = c * _NS + s

  @pl.loop(0, nchunks)
  def _(g):
    off = wid * epw + g * _B
    pltpu.sync_copy(dst_hbm.at[pl.ds(off, _B)], dst_v)
    pltpu.sync_copy(ones_v, acc_sh.at[dst_v], add=True)

  plsc.subcore_barrier()
  pltpu.sync_copy(acc_sh.at[sl], stage)
  pltpu.sync_copy(stage, out_hbm.at[pl.ds(c * np_ + s * rpt, rpt)])


def _prop_body(np_, epw, nchunks, src_hbm, dst_hbm, u0_hbm, u1_hbm, out_hbm,
               t0_sh, t1_sh, a0_sh, a1_sh, src_v, dst_v, r0_v, r1_v):
  c = lax.axis_index("c")
  s = lax.axis_index("s")
  rpt = np_ // _NS
  sl = pl.ds(s * rpt, rpt)
  stage = r0_v.at[pl.ds(0, rpt)]
  pltpu.sync_copy(u0_hbm.at[sl], stage)
  pltpu.sync_copy(stage, t0_sh.at[sl])
  pltpu.sync_copy(stage, a0_sh.at[sl])
  pltpu.sync_copy(u1_hbm.at[sl], stage)
  pltpu.sync_copy(stage, t1_sh.at[sl])
  pltpu.sync_copy(stage, a1_sh.at[sl])
  plsc.subcore_barrier()
  wid = c * _NS + s

  @pl.loop(0, nchunks)
  def _(g):
    off = wid * epw + g * _B
    pltpu.sync_copy(src_hbm.at[pl.ds(off, _B)], src_v)
    pltpu.sync_copy(dst_hbm.at[pl.ds(off, _B)], dst_v)
    pltpu.sync_copy(t0_sh.at[src_v], r0_v)
    pltpu.sync_copy(t1_sh.at[src_v], r1_v)
    pltpu.sync_copy(r0_v, a0_sh.at[dst_v], add=True)
    pltpu.sync_copy(r1_v, a1_sh.at[dst_v], add=True)

  plsc.subcore_barrier()
  base = c * 2 * np_ + s * rpt
  pltpu.sync_copy(a0_sh.at[sl], stage)
  pltpu.sync_copy(stage, out_hbm.at[pl.ds(base, rpt)])
  pltpu.sync_copy(a1_sh.at[sl], stage)
  pltpu.sync_copy(stage, out_hbm.at[pl.ds(base + np_, rpt)])


def _make_sc_kernels(np_, ep):
  epw = ep // _NW
  nchunks = epw // _B
  rpt = np_ // _NS
  buf = max(_B, rpt)
  mesh = plsc.VectorSubcoreMesh(
      core_axis_name="c", subcore_axis_name="s",
      num_cores=_NC, num_subcores=_NS)
  params = pltpu.CompilerParams(use_tc_tiling_on_sc=False)
  deg = pl.kernel(
      functools.partial(_deg_body, np_, epw, nchunks),
      out_type=jax.ShapeDtypeStruct((_NC * np_,), jnp.float32),
      mesh=mesh,
      compiler_params=params,
      scratch_types=[
          pltpu.VMEM_SHARED((np_,), jnp.float32),
          pltpu.VMEM((_B,), jnp.int32),
          pltpu.VMEM((buf,), jnp.float32),
      ],
  )
  prop = pl.kernel(
      functools.partial(_prop_body, np_, epw, nchunks),
      out_type=jax.ShapeDtypeStruct((_NC * 2 * np_,), jnp.float32),
      mesh=mesh,
      compiler_params=params,
      scratch_types=[
          pltpu.VMEM_SHARED((np_,), jnp.float32),
          pltpu.VMEM_SHARED((np_,), jnp.float32),
          pltpu.VMEM_SHARED((np_,), jnp.float32),
          pltpu.VMEM_SHARED((np_,), jnp.float32),
          pltpu.VMEM((_B,), jnp.int32),
          pltpu.VMEM((_B,), jnp.int32),
          pltpu.VMEM((buf,), jnp.float32),
          pltpu.VMEM((_B,), jnp.float32),
      ],
  )
  return deg, prop


def _prep_body(p0, p1, x0, x1, d, u0, u1):
  deg = p0[...] + p1[...] + 1.0
  dv = lax.rsqrt(deg)
  d[...] = dv
  u0[...] = dv * x0[...]
  u1[...] = dv * x1[...]


def _mid_body(a00, a01, a10, a11, u10, u11, d, w1, b1, w2,
              u20, u21):
  dv = d[...]
  p0 = dv * (a00[...] + a10[...] - u10[...])
  p1 = dv * (a01[...] + a11[...] - u11[...])
  y0 = jnp.zeros_like(p0)
  y1 = jnp.zeros_like(p0)
  for j in range(4):
    h = jnp.maximum(p0 * w1[0, j] + p1 * w1[1, j] + b1[j], 0.0)
    y0 = y0 + h * w2[j, 0]
    y1 = y1 + h * w2[j, 1]
  u20[...] = dv * y0
  u21[...] = dv * y1


def _final_body(a00, a01, a10, a11, u20, u21, d, b2, o0, o1):
  dv = d[...]
  o0[...] = dv * (a00[...] + a10[...] - u20[...]) + b2[0]
  o1[...] = dv * (a01[...] + a11[...] - u21[...]) + b2[1]


def _tc_call(body, n_in, n_smem, n_out, shape):
  sds = jax.ShapeDtypeStruct(shape, jnp.float32)
  in_specs = ([pl.BlockSpec()] * n_in
              + [pl.BlockSpec(memory_space=pltpu.SMEM)] * n_smem)
  return pl.pallas_call(
      body,
      out_shape=(sds,) * n_out,
      in_specs=in_specs,
      out_specs=(pl.BlockSpec(),) * n_out,
  )


def kernel(x, edge_index, W1, b1, W2, b2):
  n = x.shape[0]
  e = edge_index.shape[1]
  np_ = _round_up(n + 32, 128)
  ep = _round_up(e, _NW * _B)
  r = np_ // 128

  src = edge_index[0].astype(jnp.int32)
  dst = edge_index[1].astype(jnp.int32)
  if ep > e:
    pad = (jnp.arange(ep - e, dtype=jnp.int32) % (np_ - n)) + n
    src = jnp.concatenate([src, pad])
    dst = jnp.concatenate([dst, pad])

  x0 = jnp.pad(x[:, 0], (0, np_ - n)).reshape(r, 128)
  x1 = jnp.pad(x[:, 1], (0, np_ - n)).reshape(r, 128)

  deg_k, prop_k = _make_sc_kernels(np_, ep)

  degp = deg_k(dst)
  d, u0, u1 = _tc_call(_prep_body, 4, 0, 3, (r, 128))(
      degp[:np_].reshape(r, 128), degp[np_:].reshape(r, 128), x0, x1)

  acc1 = prop_k(src, dst, u0.reshape(np_), u1.reshape(np_))
  u20, u21 = _tc_call(_mid_body, 7, 3, 2, (r, 128))(
      acc1[:np_].reshape(r, 128), acc1[np_:2 * np_].reshape(r, 128),
      acc1[2 * np_:3 * np_].reshape(r, 128), acc1[3 * np_:].reshape(r, 128),
      u0, u1, d, W1, b1, W2)

  acc2 = prop_k(src, dst, u20.reshape(np_), u21.reshape(np_))
  o0, o1 = _tc_call(_final_body, 7, 1, 2, (r, 128))(
      acc2[:np_].reshape(r, 128), acc2[np_:2 * np_].reshape(r, 128),
      acc2[2 * np_:3 * np_].reshape(r, 128), acc2[3 * np_:].reshape(r, 128),
      u20, u21, d, b2)

  return jnp.stack([o0.reshape(np_)[:n], o1.reshape(np_)[:n]], axis=-1)


# speedup vs baseline: 241.4326x; 1.1283x over previous
import functools

import jax
import jax.numpy as jnp
from jax import lax
from jax.experimental import pallas as pl
from jax.experimental.pallas import tpu as pltpu
from jax.experimental.pallas import tpu_sc as plsc

_NC = 2
_NS = 16
_NW = _NC * _NS
_B = 8000


def _round_up(a, b):
  return -(-a // b) * b


def _deg_body(np_, epw, nchunks, dst_hbm, out_hbm,
              acc_sh, dst_v, ones_v):
  c = lax.axis_index("c")
  s = lax.axis_index("s")
  rpt = np_ // _NS
  sl = pl.ds(s * rpt, rpt)
  stage = ones_v.at[pl.ds(0, rpt)]

  @pl.loop(0, rpt // 16)
  def _(i):
    ones_v[pl.ds(i * 16, 16)] = jnp.zeros((16,), jnp.float32)

  pltpu.sync_copy(stage, acc_sh.at[sl])

  @pl.loop(0, _B // 16)
  def _(i):
    ones_v[pl.ds(i * 16, 16)] = jnp.full((16,), 1.0, jnp.float32)

  plsc.subcore_barrier()
  wid = c * _NS + s

  @pl.loop(0, nchunks)
  def _(g):
    off = wid * epw + g * _B
    pltpu.sync_copy(dst_hbm.at[pl.ds(off, _B)], dst_v)
    pltpu.sync_copy(ones_v, acc_sh.at[dst_v], add=True)

  plsc.subcore_barrier()
  pltpu.sync_copy(acc_sh.at[sl], stage)
  pltpu.sync_copy(stage, out_hbm.at[pl.ds(c * np_ + s * rpt, rpt)])


def _prop_body(np_, epw, nchunks, src_hbm, dst_hbm, u0_hbm, u1_hbm, out_hbm,
               t0_sh, t1_sh, a0_sh, a1_sh,
               src_v0, src_v1, src_v2, dst_v0, dst_v1, dst_v2,
               r0a_v, r0b_v, r1a_v, r1b_v, stage_v,
               isem0, isem1, isem2, gsem, ssem0, ssem1):
  c = lax.axis_index("c")
  s = lax.axis_index("s")
  rpt = np_ // _NS
  sl = pl.ds(s * rpt, rpt)
  stage = stage_v.at[pl.ds(0, rpt)]
  pltpu.sync_copy(u0_hbm.at[sl], stage)
  pltpu.sync_copy(stage, t0_sh.at[sl])
  pltpu.sync_copy(stage, a0_sh.at[sl])
  pltpu.sync_copy(u1_hbm.at[sl], stage)
  pltpu.sync_copy(stage, t1_sh.at[sl])
  pltpu.sync_copy(stage, a1_sh.at[sl])
  plsc.subcore_barrier()
  wid = c * _NS + s

  srcs = (src_v0, src_v1, src_v2)
  dsts = (dst_v0, dst_v1, dst_v2)
  r0s = (r0a_v, r0b_v)
  r1s = (r1a_v, r1b_v)
  isems = (isem0, isem1, isem2)
  ssems = (ssem0, ssem1)

  def idx_slice(g):
    return pl.ds(wid * epw + g * _B, _B)

  def issue_idx(g):
    b = g % 3
    pltpu.async_copy(src_hbm.at[idx_slice(g)], srcs[b], isems[b])
    pltpu.async_copy(dst_hbm.at[idx_slice(g)], dsts[b], isems[b])

  def wait_idx(g):
    b = g % 3
    pltpu.make_async_copy(src_hbm.at[idx_slice(g)], srcs[b], isems[b]).wait()
    pltpu.make_async_copy(dst_hbm.at[idx_slice(g)], dsts[b], isems[b]).wait()

  def drain_scatter(g):
    b2, b3 = g % 2, g % 3
    pltpu.make_async_copy(r0s[b2], a0_sh.at[dsts[b3]], ssems[b2]).wait()
    pltpu.make_async_copy(r1s[b2], a1_sh.at[dsts[b3]], ssems[b2]).wait()

  issue_idx(0)
  for g in range(nchunks):
    b2, b3 = g % 2, g % 3
    wait_idx(g)
    if g >= 2:
      drain_scatter(g - 2)
    if g + 1 < nchunks:
      issue_idx(g + 1)
    d0 = pltpu.async_copy(t0_sh.at[srcs[b3]], r0s[b2], gsem)
    d1 = pltpu.async_copy(t1_sh.at[srcs[b3]], r1s[b2], gsem)
    d0.wait()
    d1.wait()
    pltpu.async_copy(r0s[b2], a0_sh.at[dsts[b3]], ssems[b2], add=True)
    pltpu.async_copy(r1s[b2], a1_sh.at[dsts[b3]], ssems[b2], add=True)
  for g in range(max(0, nchunks - 2), nchunks):
    drain_scatter(g)

  plsc.subcore_barrier()
  base = c * 2 * np_ + s * rpt
  pltpu.sync_copy(a0_sh.at[sl], stage)
  pltpu.sync_copy(stage, out_hbm.at[pl.ds(base, rpt)])
  pltpu.sync_copy(a1_sh.at[sl], stage)
  pltpu.sync_copy(stage, out_hbm.at[pl.ds(base + np_, rpt)])


def _make_sc_kernels(np_, ep):
  epw = ep // _NW
  nchunks = epw // _B
  rpt = np_ // _NS
  buf = max(_B, rpt)
  mesh = plsc.VectorSubcoreMesh(
      core_axis_name="c", subcore_axis_name="s",
      num_cores=_NC, num_subcores=_NS)
  params = pltpu.CompilerParams(use_tc_tiling_on_sc=False)
  deg = pl.kernel(
      functools.partial(_deg_body, np_, epw, nchunks),
      out_type=jax.ShapeDtypeStruct((_NC * np_,), jnp.float32),
      mesh=mesh,
      compiler_params=params,
      scratch_types=[
          pltpu.VMEM_SHARED((np_,), jnp.float32),
          pltpu.VMEM((_B,), jnp.int32),
          pltpu.VMEM((buf,), jnp.float32),
      ],
  )
  prop = pl.kernel(
      functools.partial(_prop_body, np_, epw, nchunks),
      out_type=jax.ShapeDtypeStruct((_NC * 2 * np_,), jnp.float32),
      mesh=mesh,
      compiler_params=params,
      scratch_types=[
          pltpu.VMEM_SHARED((np_,), jnp.float32),
          pltpu.VMEM_SHARED((np_,), jnp.float32),
          pltpu.VMEM_SHARED((np_,), jnp.float32),
          pltpu.VMEM_SHARED((np_,), jnp.float32),
          pltpu.VMEM((_B,), jnp.int32),
          pltpu.VMEM((_B,), jnp.int32),
          pltpu.VMEM((_B,), jnp.int32),
          pltpu.VMEM((_B,), jnp.int32),
          pltpu.VMEM((_B,), jnp.int32),
          pltpu.VMEM((_B,), jnp.int32),
          pltpu.VMEM((_B,), jnp.float32),
          pltpu.VMEM((_B,), jnp.float32),
          pltpu.VMEM((_B,), jnp.float32),
          pltpu.VMEM((_B,), jnp.float32),
          pltpu.VMEM((buf,), jnp.float32),
          pltpu.SemaphoreType.DMA,
          pltpu.SemaphoreType.DMA,
          pltpu.SemaphoreType.DMA,
          pltpu.SemaphoreType.DMA,
          pltpu.SemaphoreType.DMA,
          pltpu.SemaphoreType.DMA,
      ],
  )
  return deg, prop


def _prep_body(p0, p1, x0, x1, d, u0, u1):
  deg = p0[...] + p1[...] + 1.0
  dv = lax.rsqrt(deg)
  d[...] = dv
  u0[...] = dv * x0[...]
  u1[...] = dv * x1[...]


def _mid_body(a00, a01, a10, a11, u10, u11, d, w1, b1, w2,
              u20, u21):
  dv = d[...]
  p0 = dv * (a00[...] + a10[...] - u10[...])
  p1 = dv * (a01[...] + a11[...] - u11[...])
  y0 = jnp.zeros_like(p0)
  y1 = jnp.zeros_like(p0)
  for j in range(4):
    h = jnp.maximum(p0 * w1[0, j] + p1 * w1[1, j] + b1[j], 0.0)
    y0 = y0 + h * w2[j, 0]
    y1 = y1 + h * w2[j, 1]
  u20[...] = dv * y0
  u21[...] = dv * y1


def _final_body(a00, a01, a10, a11, u20, u21, d, b2, o0, o1):
  dv = d[...]
  o0[...] = dv * (a00[...] + a10[...] - u20[...]) + b2[0]
  o1[...] = dv * (a01[...] + a11[...] - u21[...]) + b2[1]


def _tc_call(body, n_in, n_smem, n_out, shape):
  sds = jax.ShapeDtypeStruct(shape, jnp.float32)
  in_specs = ([pl.BlockSpec()] * n_in
              + [pl.BlockSpec(memory_space=pltpu.SMEM)] * n_smem)
  return pl.pallas_call(
      body,
      out_shape=(sds,) * n_out,
      in_specs=in_specs,
      out_specs=(pl.BlockSpec(),) * n_out,
  )


def kernel(x, edge_index, W1, b1, W2, b2):
  n = x.shape[0]
  e = edge_index.shape[1]
  np_ = _round_up(n + 32, 128)
  ep = _round_up(e, _NW * _B)
  r = np_ // 128

  src = edge_index[0].astype(jnp.int32)
  dst = edge_index[1].astype(jnp.int32)
  if ep > e:
    pad = (jnp.arange(ep - e, dtype=jnp.int32) % (np_ - n)) + n
    src = jnp.concatenate([src, pad])
    dst = jnp.concatenate([dst, pad])

  x0 = jnp.pad(x[:, 0], (0, np_ - n)).reshape(r, 128)
  x1 = jnp.pad(x[:, 1], (0, np_ - n)).reshape(r, 128)

  deg_k, prop_k = _make_sc_kernels(np_, ep)

  degp = deg_k(dst)
  d, u0, u1 = _tc_call(_prep_body, 4, 0, 3, (r, 128))(
      degp[:np_].reshape(r, 128), degp[np_:].reshape(r, 128), x0, x1)

  acc1 = prop_k(src, dst, u0.reshape(np_), u1.reshape(np_))
  u20, u21 = _tc_call(_mid_body, 7, 3, 2, (r, 128))(
      acc1[:np_].reshape(r, 128), acc1[np_:2 * np_].reshape(r, 128),
      acc1[2 * np_:3 * np_].reshape(r, 128), acc1[3 * np_:].reshape(r, 128),
      u0, u1, d, W1, b1, W2)

  acc2 = prop_k(src, dst, u20.reshape(np_), u21.reshape(np_))
  o0, o1 = _tc_call(_final_body, 7, 1, 2, (r, 128))(
      acc2[:np_].reshape(r, 128), acc2[np_:2 * np_].reshape(r, 128),
      acc2[2 * np_:3 * np_].reshape(r, 128), acc2[3 * np_:].reshape(r, 128),
      u20, u21, d, b2)

  return jnp.stack([o0.reshape(np_)[:n], o1.reshape(np_)[:n]], axis=-1)


# speedup vs baseline: 242.2786x; 1.0035x over previous
import functools

import jax
import jax.numpy as jnp
from jax import lax
from jax.experimental import pallas as pl
from jax.experimental.pallas import tpu as pltpu
from jax.experimental.pallas import tpu_sc as plsc

_NC = 2
_NS = 16
_NW = _NC * _NS
_B = 4000
_BD = 8000


def _round_up(a, b):
  return -(-a // b) * b


def _deg_body(np_, epw, nchunks, dst_hbm, out_hbm,
              acc_sh, dst_v, ones_v):
  c = lax.axis_index("c")
  s = lax.axis_index("s")
  rpt = np_ // _NS
  sl = pl.ds(s * rpt, rpt)
  stage = ones_v.at[pl.ds(0, rpt)]

  @pl.loop(0, rpt // 16)
  def _(i):
    ones_v[pl.ds(i * 16, 16)] = jnp.zeros((16,), jnp.float32)

  pltpu.sync_copy(stage, acc_sh.at[sl])

  @pl.loop(0, _BD // 16)
  def _(i):
    ones_v[pl.ds(i * 16, 16)] = jnp.full((16,), 1.0, jnp.float32)

  plsc.subcore_barrier()
  wid = c * _NS + s

  @pl.loop(0, nchunks)
  def _(g):
    off = wid * epw + g * _BD
    pltpu.sync_copy(dst_hbm.at[pl.ds(off, _BD)], dst_v)
    pltpu.sync_copy(ones_v.at[pl.ds(0, _BD)], acc_sh.at[dst_v], add=True)

  plsc.subcore_barrier()
  pltpu.sync_copy(acc_sh.at[sl], stage)
  pltpu.sync_copy(stage, out_hbm.at[pl.ds(c * np_ + s * rpt, rpt)])


def _prop_body(np_, epw, nchunks, src_hbm, dst_hbm, w_hbm, u0_hbm, u1_hbm,
               out_hbm,
               t_sh, a0_sh, a1_sh,
               src_v0, src_v1, src_v2, dst_v0, dst_v1, dst_v2,
               gwa_v, gwb_v, r0a_v, r0b_v, r1a_v, r1b_v, stage_v, tstage_v,
               isem0, isem1, isem2, gsem0, gsem1, ssem0, ssem1):
  c = lax.axis_index("c")
  s = lax.axis_index("s")
  rpt = np_ // _NS
  sl = pl.ds(s * rpt, rpt)
  stage = stage_v.at[pl.ds(0, rpt)]
  pltpu.sync_copy(u0_hbm.at[sl], stage)
  pltpu.sync_copy(stage, a0_sh.at[sl])
  pltpu.sync_copy(u1_hbm.at[sl], stage)
  pltpu.sync_copy(stage, a1_sh.at[sl])
  pltpu.sync_copy(w_hbm.at[sl], tstage_v)
  pltpu.sync_copy(tstage_v, t_sh.at[sl])
  plsc.subcore_barrier()
  wid = c * _NS + s

  srcs = (src_v0, src_v1, src_v2)
  dsts = (dst_v0, dst_v1, dst_v2)
  gws = (gwa_v, gwb_v)
  r0s = (r0a_v, r0b_v)
  r1s = (r1a_v, r1b_v)
  isems = (isem0, isem1, isem2)
  gsems = (gsem0, gsem1)
  ssems = (ssem0, ssem1)

  def idx_slice(g):
    return pl.ds(wid * epw + g * _B, _B)

  def issue_idx(g):
    b = g % 3
    pltpu.async_copy(src_hbm.at[idx_slice(g)], srcs[b], isems[b])
    pltpu.async_copy(dst_hbm.at[idx_slice(g)], dsts[b], isems[b])

  def wait_idx(g):
    b = g % 3
    pltpu.make_async_copy(src_hbm.at[idx_slice(g)], srcs[b], isems[b]).wait()
    pltpu.make_async_copy(dst_hbm.at[idx_slice(g)], dsts[b], isems[b]).wait()

  def issue_gather(g):
    pltpu.async_copy(t_sh.at[srcs[g % 3]], gws[g % 2], gsems[g % 2])

  def wait_gather(g):
    pltpu.make_async_copy(t_sh.at[srcs[g % 3]], gws[g % 2],
                          gsems[g % 2]).wait()

  def drain_scatter(g):
    b2, b3 = g % 2, g % 3
    pltpu.make_async_copy(r0s[b2], a0_sh.at[dsts[b3]], ssems[b2]).wait()
    pltpu.make_async_copy(r1s[b2], a1_sh.at[dsts[b3]], ssems[b2]).wait()

  def process(g):
    b2, b3 = g % 2, g % 3
    wait_gather(g)
    gw_b = gws[b2]
    r0_b = r0s[b2]
    r1_b = r1s[b2]

    @pl.loop(0, _B // 16, unroll=4)
    def _(i):
      ii = pl.ds(i * 16, 16)
      w = gw_b[ii]
      sh = jnp.full((16,), 16, jnp.int32)
      msk = jnp.full((16,), -65536, jnp.int32)
      r0_b[ii] = lax.bitcast_convert_type(w << sh, jnp.float32)
      r1_b[ii] = lax.bitcast_convert_type(w & msk, jnp.float32)

    pltpu.async_copy(r0_b, a0_sh.at[dsts[b3]], ssems[b2], add=True)
    pltpu.async_copy(r1_b, a1_sh.at[dsts[b3]], ssems[b2], add=True)

  issue_idx(0)
  for g in range(nchunks):
    wait_idx(g)
    issue_gather(g)
    if g >= 2:
      drain_scatter(g - 2)
    if g + 1 < nchunks:
      issue_idx(g + 1)
    if g >= 1:
      process(g - 1)
  process(nchunks - 1)
  for g in range(max(0, nchunks - 2), nchunks):
    drain_scatter(g)

  plsc.subcore_barrier()
  base = c * 2 * np_ + s * rpt
  pltpu.sync_copy(a0_sh.at[sl], stage)
  pltpu.sync_copy(stage, out_hbm.at[pl.ds(base, rpt)])
  pltpu.sync_copy(a1_sh.at[sl], stage)
  pltpu.sync_copy(stage, out_hbm.at[pl.ds(base + np_, rpt)])


def _make_sc_kernels(np_, ep):
  epw = ep // _NW
  nchunks = epw // _B
  nchunks_d = epw // _BD
  rpt = np_ // _NS
  mesh = plsc.VectorSubcoreMesh(
      core_axis_name="c", subcore_axis_name="s",
      num_cores=_NC, num_subcores=_NS)
  params = pltpu.CompilerParams(use_tc_tiling_on_sc=False)
  deg = pl.kernel(
      functools.partial(_deg_body, np_, epw, nchunks_d),
      out_type=jax.ShapeDtypeStruct((_NC * np_,), jnp.float32),
      mesh=mesh,
      compiler_params=params,
      scratch_types=[
          pltpu.VMEM_SHARED((np_,), jnp.float32),
          pltpu.VMEM((_BD,), jnp.int32),
          pltpu.VMEM((max(_BD, rpt),), jnp.float32),
      ],
  )
  prop = pl.kernel(
      functools.partial(_prop_body, np_, epw, nchunks),
      out_type=jax.ShapeDtypeStruct((_NC * 2 * np_,), jnp.float32),
      mesh=mesh,
      compiler_params=params,
      scratch_types=[
          pltpu.VMEM_SHARED((np_,), jnp.int32),
          pltpu.VMEM_SHARED((np_,), jnp.float32),
          pltpu.VMEM_SHARED((np_,), jnp.float32),
          pltpu.VMEM((_B,), jnp.int32),
          pltpu.VMEM((_B,), jnp.int32),
          pltpu.VMEM((_B,), jnp.int32),
          pltpu.VMEM((_B,), jnp.int32),
          pltpu.VMEM((_B,), jnp.int32),
          pltpu.VMEM((_B,), jnp.int32),
          pltpu.VMEM((_B,), jnp.int32),
          pltpu.VMEM((_B,), jnp.int32),
          pltpu.VMEM((_B,), jnp.float32),
          pltpu.VMEM((_B,), jnp.float32),
          pltpu.VMEM((_B,), jnp.float32),
          pltpu.VMEM((_B,), jnp.float32),
          pltpu.VMEM((rpt,), jnp.float32),
          pltpu.VMEM((rpt,), jnp.int32),
          pltpu.SemaphoreType.DMA,
          pltpu.SemaphoreType.DMA,
          pltpu.SemaphoreType.DMA,
          pltpu.SemaphoreType.DMA,
          pltpu.SemaphoreType.DMA,
          pltpu.SemaphoreType.DMA,
          pltpu.SemaphoreType.DMA,
      ],
  )
  return deg, prop


def _pack_cols(u0v, u1v):
  b0 = lax.bitcast_convert_type(u0v.astype(jnp.bfloat16), jnp.uint16)
  b1 = lax.bitcast_convert_type(u1v.astype(jnp.bfloat16), jnp.uint16)
  w = (b1.astype(jnp.uint32) << 16) | b0.astype(jnp.uint32)
  return lax.bitcast_convert_type(w, jnp.int32)


def _prep_body(p0, p1, x0, x1, d, u0, u1, w):
  deg = p0[...] + p1[...] + 1.0
  dv = lax.rsqrt(deg)
  d[...] = dv
  u0v = dv * x0[...]
  u1v = dv * x1[...]
  u0[...] = u0v
  u1[...] = u1v
  w[...] = _pack_cols(u0v, u1v)


def _mid_body(a00, a01, a10, a11, u10, u11, d, w1, b1, w2,
              u20, u21, w):
  dv = d[...]
  p0 = dv * (a00[...] + a10[...] - u10[...])
  p1 = dv * (a01[...] + a11[...] - u11[...])
  y0 = jnp.zeros_like(p0)
  y1 = jnp.zeros_like(p0)
  for j in range(4):
    h = jnp.maximum(p0 * w1[0, j] + p1 * w1[1, j] + b1[j], 0.0)
    y0 = y0 + h * w2[j, 0]
    y1 = y1 + h * w2[j, 1]
  u20v = dv * y0
  u21v = dv * y1
  u20[...] = u20v
  u21[...] = u21v
  w[...] = _pack_cols(u20v, u21v)


def _final_body(a00, a01, a10, a11, u20, u21, d, b2, o0, o1):
  dv = d[...]
  o0[...] = dv * (a00[...] + a10[...] - u20[...]) + b2[0]
  o1[...] = dv * (a01[...] + a11[...] - u21[...]) + b2[1]


def _tc_call(body, n_in, n_smem, out_dtypes, shape):
  in_specs = ([pl.BlockSpec()] * n_in
              + [pl.BlockSpec(memory_space=pltpu.SMEM)] * n_smem)
  return pl.pallas_call(
      body,
      out_shape=tuple(jax.ShapeDtypeStruct(shape, dt) for dt in out_dtypes),
      in_specs=in_specs,
      out_specs=(pl.BlockSpec(),) * len(out_dtypes),
  )


def kernel(x, edge_index, W1, b1, W2, b2):
  n = x.shape[0]
  e = edge_index.shape[1]
  np_ = _round_up(n + 32, 128)
  ep = _round_up(e, _NW * _BD)
  r = np_ // 128

  src = edge_index[0].astype(jnp.int32)
  dst = edge_index[1].astype(jnp.int32)
  if ep > e:
    pad = (jnp.arange(ep - e, dtype=jnp.int32) % (np_ - n)) + n
    src = jnp.concatenate([src, pad])
    dst = jnp.concatenate([dst, pad])

  x0 = jnp.pad(x[:, 0], (0, np_ - n)).reshape(r, 128)
  x1 = jnp.pad(x[:, 1], (0, np_ - n)).reshape(r, 128)

  deg_k, prop_k = _make_sc_kernels(np_, ep)

  f32 = jnp.float32
  degp = deg_k(dst)
  d, u0, u1, w1p = _tc_call(_prep_body, 4, 0, (f32, f32, f32, jnp.int32),
                            (r, 128))(
      degp[:np_].reshape(r, 128), degp[np_:].reshape(r, 128), x0, x1)

  acc1 = prop_k(src, dst, w1p.reshape(np_),
                u0.reshape(np_), u1.reshape(np_))
  u20, u21, w2p = _tc_call(_mid_body, 7, 3, (f32, f32, jnp.int32),
                           (r, 128))(
      acc1[:np_].reshape(r, 128), acc1[np_:2 * np_].reshape(r, 128),
      acc1[2 * np_:3 * np_].reshape(r, 128), acc1[3 * np_:].reshape(r, 128),
      u0, u1, d, W1, b1, W2)

  acc2 = prop_k(src, dst, w2p.reshape(np_),
                u20.reshape(np_), u21.reshape(np_))
  o0, o1 = _tc_call(_final_body, 7, 1, (f32, f32), (r, 128))(
      acc2[:np_].reshape(r, 128), acc2[np_:2 * np_].reshape(r, 128),
      acc2[2 * np_:3 * np_].reshape(r, 128), acc2[3 * np_:].reshape(r, 128),
      u20, u21, d, b2)

  return jnp.stack([o0.reshape(np_)[:n], o1.reshape(np_)[:n]], axis=-1)


# speedup vs baseline: 246.9993x; 1.0195x over previous
import functools

import jax
import jax.numpy as jnp
from jax import lax
from jax.experimental import pallas as pl
from jax.experimental.pallas import tpu as pltpu
from jax.experimental.pallas import tpu_sc as plsc

_NC = 2
_NS = 16
_NW = _NC * _NS
_B = 8000
_BD = 8000


def _round_up(a, b):
  return -(-a // b) * b


def _deg_body(np_, epw, nchunks, dst_hbm, out_hbm,
              acc_sh, dst_v, ones_v):
  c = lax.axis_index("c")
  s = lax.axis_index("s")
  rpt = np_ // _NS
  sl = pl.ds(s * rpt, rpt)
  stage = ones_v.at[pl.ds(0, rpt)]

  @pl.loop(0, rpt // 16)
  def _(i):
    ones_v[pl.ds(i * 16, 16)] = jnp.zeros((16,), jnp.float32)

  pltpu.sync_copy(stage, acc_sh.at[sl])

  @pl.loop(0, _BD // 16)
  def _(i):
    ones_v[pl.ds(i * 16, 16)] = jnp.full((16,), 1.0, jnp.float32)

  plsc.subcore_barrier()
  wid = c * _NS + s

  @pl.loop(0, nchunks)
  def _(g):
    off = wid * epw + g * _BD
    pltpu.sync_copy(dst_hbm.at[pl.ds(off, _BD)], dst_v)
    pltpu.sync_copy(ones_v.at[pl.ds(0, _BD)], acc_sh.at[dst_v], add=True)

  plsc.subcore_barrier()
  pltpu.sync_copy(acc_sh.at[sl], stage)
  pltpu.sync_copy(stage, out_hbm.at[pl.ds(c * np_ + s * rpt, rpt)])


def _prop_body(np_, epw, nchunks, src_hbm, dst_hbm, w_hbm, u0_hbm, u1_hbm,
               out_hbm,
               t_sh, a0_sh, a1_sh,
               src_v0, src_v1, src_v2, dst_v0, dst_v1, dst_v2,
               gwa_v, gwb_v, r0a_v, r0b_v, r1a_v, r1b_v,
               isem0, isem1, isem2, gsem0, gsem1, ssem0, ssem1):
  c = lax.axis_index("c")
  s = lax.axis_index("s")
  rpt = np_ // _NS
  sl = pl.ds(s * rpt, rpt)
  stage = r0a_v.at[pl.ds(0, rpt)]
  tstage = gwa_v.at[pl.ds(0, rpt)]
  pltpu.sync_copy(u0_hbm.at[sl], stage)
  pltpu.sync_copy(stage, a0_sh.at[sl])
  pltpu.sync_copy(u1_hbm.at[sl], stage)
  pltpu.sync_copy(stage, a1_sh.at[sl])
  pltpu.sync_copy(w_hbm.at[sl], tstage)
  pltpu.sync_copy(tstage, t_sh.at[sl])
  plsc.subcore_barrier()
  wid = c * _NS + s

  srcs = (src_v0, src_v1, src_v2)
  dsts = (dst_v0, dst_v1, dst_v2)
  gws = (gwa_v, gwb_v)
  r0s = (r0a_v, r0b_v)
  r1s = (r1a_v, r1b_v)
  isems = (isem0, isem1, isem2)
  gsems = (gsem0, gsem1)
  ssems = (ssem0, ssem1)

  def idx_slice(g):
    return pl.ds(wid * epw + g * _B, _B)

  def issue_idx(g):
    b = g % 3
    pltpu.async_copy(src_hbm.at[idx_slice(g)], srcs[b], isems[b])
    pltpu.async_copy(dst_hbm.at[idx_slice(g)], dsts[b], isems[b])

  def wait_idx(g):
    b = g % 3
    pltpu.make_async_copy(src_hbm.at[idx_slice(g)], srcs[b], isems[b]).wait()
    pltpu.make_async_copy(dst_hbm.at[idx_slice(g)], dsts[b], isems[b]).wait()

  def issue_gather(g):
    pltpu.async_copy(t_sh.at[srcs[g % 3]], gws[g % 2], gsems[g % 2])

  def wait_gather(g):
    pltpu.make_async_copy(t_sh.at[srcs[g % 3]], gws[g % 2],
                          gsems[g % 2]).wait()

  def drain_scatter(g):
    b2, b3 = g % 2, g % 3
    pltpu.make_async_copy(r0s[b2], a0_sh.at[dsts[b3]], ssems[b2]).wait()
    pltpu.make_async_copy(r1s[b2], a1_sh.at[dsts[b3]], ssems[b2]).wait()

  def process(g):
    b2, b3 = g % 2, g % 3
    wait_gather(g)
    gw_b = gws[b2]
    r0_b = r0s[b2]
    r1_b = r1s[b2]

    @pl.loop(0, _B // 16, unroll=4)
    def _(i):
      ii = pl.ds(i * 16, 16)
      w = gw_b[ii]
      sh = jnp.full((16,), 16, jnp.int32)
      msk = jnp.full((16,), -65536, jnp.int32)
      r0_b[ii] = lax.bitcast_convert_type(w << sh, jnp.float32)
      r1_b[ii] = lax.bitcast_convert_type(w & msk, jnp.float32)

    pltpu.async_copy(r0_b, a0_sh.at[dsts[b3]], ssems[b2], add=True)
    pltpu.async_copy(r1_b, a1_sh.at[dsts[b3]], ssems[b2], add=True)

  issue_idx(0)
  for g in range(nchunks):
    wait_idx(g)
    issue_gather(g)
    if g >= 2:
      drain_scatter(g - 2)
    if g + 1 < nchunks:
      issue_idx(g + 1)
    if g >= 1:
      process(g - 1)
  process(nchunks - 1)
  for g in range(max(0, nchunks - 2), nchunks):
    drain_scatter(g)

  plsc.subcore_barrier()
  base = c * 2 * np_ + s * rpt
  pltpu.sync_copy(a0_sh.at[sl], stage)
  pltpu.sync_copy(stage, out_hbm.at[pl.ds(base, rpt)])
  pltpu.sync_copy(a1_sh.at[sl], stage)
  pltpu.sync_copy(stage, out_hbm.at[pl.ds(base + np_, rpt)])


def _make_sc_kernels(np_, ep):
  epw = ep // _NW
  nchunks = epw // _B
  nchunks_d = epw // _BD
  rpt = np_ // _NS
  mesh = plsc.VectorSubcoreMesh(
      core_axis_name="c", subcore_axis_name="s",
      num_cores=_NC, num_subcores=_NS)
  params = pltpu.CompilerParams(use_tc_tiling_on_sc=False)
  deg = pl.kernel(
      functools.partial(_deg_body, np_, epw, nchunks_d),
      out_type=jax.ShapeDtypeStruct((_NC * np_,), jnp.float32),
      mesh=mesh,
      compiler_params=params,
      scratch_types=[
          pltpu.VMEM_SHARED((np_,), jnp.float32),
          pltpu.VMEM((_BD,), jnp.int32),
          pltpu.VMEM((max(_BD, rpt),), jnp.float32),
      ],
  )
  prop = pl.kernel(
      functools.partial(_prop_body, np_, epw, nchunks),
      out_type=jax.ShapeDtypeStruct((_NC * 2 * np_,), jnp.float32),
      mesh=mesh,
      compiler_params=params,
      scratch_types=[
          pltpu.VMEM_SHARED((np_,), jnp.int32),
          pltpu.VMEM_SHARED((np_,), jnp.float32),
          pltpu.VMEM_SHARED((np_,), jnp.float32),
          pltpu.VMEM((_B,), jnp.int32),
          pltpu.VMEM((_B,), jnp.int32),
          pltpu.VMEM((_B,), jnp.int32),
          pltpu.VMEM((_B,), jnp.int32),
          pltpu.VMEM((_B,), jnp.int32),
          pltpu.VMEM((_B,), jnp.int32),
          pltpu.VMEM((_B,), jnp.int32),
          pltpu.VMEM((_B,), jnp.int32),
          pltpu.VMEM((_B,), jnp.float32),
          pltpu.VMEM((_B,), jnp.float32),
          pltpu.VMEM((_B,), jnp.float32),
          pltpu.VMEM((_B,), jnp.float32),
          pltpu.SemaphoreType.DMA,
          pltpu.SemaphoreType.DMA,
          pltpu.SemaphoreType.DMA,
          pltpu.SemaphoreType.DMA,
          pltpu.SemaphoreType.DMA,
          pltpu.SemaphoreType.DMA,
          pltpu.SemaphoreType.DMA,
      ],
  )
  return deg, prop


def _pack_cols(u0v, u1v):
  b0 = lax.bitcast_convert_type(u0v.astype(jnp.bfloat16), jnp.uint16)
  b1 = lax.bitcast_convert_type(u1v.astype(jnp.bfloat16), jnp.uint16)
  w = (b1.astype(jnp.uint32) << 16) | b0.astype(jnp.uint32)
  return lax.bitcast_convert_type(w, jnp.int32)


def _prep_body(p0, p1, x0, x1, d, u0, u1, w):
  deg = p0[...] + p1[...] + 1.0
  dv = lax.rsqrt(deg)
  d[...] = dv
  u0v = dv * x0[...]
  u1v = dv * x1[...]
  u0[...] = u0v
  u1[...] = u1v
  w[...] = _pack_cols(u0v, u1v)


def _mid_body(a00, a01, a10, a11, u10, u11, d, w1, b1, w2,
              u20, u21, w):
  dv = d[...]
  p0 = dv * (a00[...] + a10[...] - u10[...])
  p1 = dv * (a01[...] + a11[...] - u11[...])
  y0 = jnp.zeros_like(p0)
  y1 = jnp.zeros_like(p0)
  for j in range(4):
    h = jnp.maximum(p0 * w1[0, j] + p1 * w1[1, j] + b1[j], 0.0)
    y0 = y0 + h * w2[j, 0]
    y1 = y1 + h * w2[j, 1]
  u20v = dv * y0
  u21v = dv * y1
  u20[...] = u20v
  u21[...] = u21v
  w[...] = _pack_cols(u20v, u21v)


def _final_body(a00, a01, a10, a11, u20, u21, d, b2, o0, o1):
  dv = d[...]
  o0[...] = dv * (a00[...] + a10[...] - u20[...]) + b2[0]
  o1[...] = dv * (a01[...] + a11[...] - u21[...]) + b2[1]


def _tc_call(body, n_in, n_smem, out_dtypes, shape):
  in_specs = ([pl.BlockSpec()] * n_in
              + [pl.BlockSpec(memory_space=pltpu.SMEM)] * n_smem)
  return pl.pallas_call(
      body,
      out_shape=tuple(jax.ShapeDtypeStruct(shape, dt) for dt in out_dtypes),
      in_specs=in_specs,
      out_specs=(pl.BlockSpec(),) * len(out_dtypes),
  )


def kernel(x, edge_index, W1, b1, W2, b2):
  n = x.shape[0]
  e = edge_index.shape[1]
  np_ = _round_up(n + 32, 128)
  ep = _round_up(e, _NW * _BD)
  r = np_ // 128

  src = edge_index[0].astype(jnp.int32)
  dst = edge_index[1].astype(jnp.int32)
  if ep > e:
    pad = (jnp.arange(ep - e, dtype=jnp.int32) % (np_ - n)) + n
    src = jnp.concatenate([src, pad])
    dst = jnp.concatenate([dst, pad])

  x0 = jnp.pad(x[:, 0], (0, np_ - n)).reshape(r, 128)
  x1 = jnp.pad(x[:, 1], (0, np_ - n)).reshape(r, 128)

  deg_k, prop_k = _make_sc_kernels(np_, ep)

  f32 = jnp.float32
  degp = deg_k(dst)
  d, u0, u1, w1p = _tc_call(_prep_body, 4, 0, (f32, f32, f32, jnp.int32),
                            (r, 128))(
      degp[:np_].reshape(r, 128), degp[np_:].reshape(r, 128), x0, x1)

  acc1 = prop_k(src, dst, w1p.reshape(np_),
                u0.reshape(np_), u1.reshape(np_))
  u20, u21, w2p = _tc_call(_mid_body, 7, 3, (f32, f32, jnp.int32),
                           (r, 128))(
      acc1[:np_].reshape(r, 128), acc1[np_:2 * np_].reshape(r, 128),
      acc1[2 * np_:3 * np_].reshape(r, 128), acc1[3 * np_:].reshape(r, 128),
      u0, u1, d, W1, b1, W2)

  acc2 = prop_k(src, dst, w2p.reshape(np_),
                u20.reshape(np_), u21.reshape(np_))
  o0, o1 = _tc_call(_final_body, 7, 1, (f32, f32), (r, 128))(
      acc2[:np_].reshape(r, 128), acc2[np_:2 * np_].reshape(r, 128),
      acc2[2 * np_:3 * np_].reshape(r, 128), acc2[3 * np_:].reshape(r, 128),
      u20, u21, d, b2)

  return jnp.stack([o0.reshape(np_)[:n], o1.reshape(np_)[:n]], axis=-1)
